# baseline (speedup 1.0000x reference)
import jax
import jax.numpy as jnp
from jax import lax
from jax.experimental import pallas as pl
from jax.experimental.pallas import tpu as pltpu
from jax.experimental.pallas import tpu_sc as plsc

B = 4
N = 160000
D = 128
S = 10000
NC = 2
NS = 16

C = 80
CPT = N // (NS * C)
ROWS_PER_TILE = C * CPT
IDROWS = N // C
SEG_PER_TILE = S // NS
ZROWS = 25


def _seg_body(data_hbm, ids_hbm, out_hbm, idx_v, chunk_a, chunk_b, chunk_c,
              zero_v, acc_sh, sem_la, sem_lb, sem_lc, sem_i, sem_z):
    c = lax.axis_index("c")
    s = lax.axis_index("s")

    def _zfill(k, carry):
        zero_v[k // (D // 16), pl.ds((k % (D // 16)) * 16, 16)] = jnp.zeros(
            (16,), jnp.float32)
        return carry
    lax.fori_loop(0, ZROWS * (D // 16), _zfill, 0)

    row0 = s * ROWS_PER_TILE
    idrow0 = s * CPT

    for step in range(B // NC):
        batch = c * (B // NC) + step

        def _load(j, buf, sem):
            pltpu.async_copy(
                data_hbm.at[batch, pl.ds(row0 + j * C, C)], buf, sem)

        def _wait_load(buf, sem):
            pltpu.make_async_copy(
                data_hbm.at[batch, pl.ds(row0, C)], buf, sem).wait()

        def _scat(j, buf):
            pltpu.sync_copy(buf, acc_sh.at[idx_v.at[j]], add=True)

        ids_d = pltpu.async_copy(ids_hbm.at[batch, pl.ds(idrow0, CPT)], idx_v,
                                 sem_i)
        zero_d = [
            pltpu.async_copy(
                zero_v, acc_sh.at[pl.ds(s * SEG_PER_TILE + k * ZROWS, ZROWS)],
                sem_z)
            for k in range(SEG_PER_TILE // ZROWS)
        ]
        _load(0, chunk_a, sem_la)
        _load(1, chunk_b, sem_lb)
        for d in zero_d:
            d.wait()
        ids_d.wait()
        plsc.subcore_barrier()

        _wait_load(chunk_a, sem_la)
        _load(2, chunk_c, sem_lc)
        _scat(0, chunk_a)
        _wait_load(chunk_b, sem_lb)
        _load(3, chunk_a, sem_la)
        _scat(1, chunk_b)

        def _tri(t, carry):
            j = 3 * t + 2
            _wait_load(chunk_c, sem_lc)
            _load(j + 2, chunk_b, sem_lb)
            _scat(j, chunk_c)

            _wait_load(chunk_a, sem_la)
            _load(j + 3, chunk_c, sem_lc)
            _scat(j + 1, chunk_a)

            _wait_load(chunk_b, sem_lb)
            _load(j + 4, chunk_a, sem_la)
            _scat(j + 2, chunk_b)
            return carry
        lax.fori_loop(0, (CPT - 5) // 3, _tri, 0)

        _wait_load(chunk_c, sem_lc)
        _load(CPT - 1, chunk_b, sem_lb)
        _scat(CPT - 3, chunk_c)

        _wait_load(chunk_a, sem_la)
        _scat(CPT - 2, chunk_a)

        _wait_load(chunk_b, sem_lb)
        _scat(CPT - 1, chunk_b)
        plsc.subcore_barrier()

        pltpu.sync_copy(
            acc_sh.at[pl.ds(s * SEG_PER_TILE, SEG_PER_TILE)],
            out_hbm.at[batch, pl.ds(s * SEG_PER_TILE, SEG_PER_TILE)])
        plsc.subcore_barrier()


@jax.jit
def kernel(data, segment_ids):
    ids32 = segment_ids.astype(jnp.int32).reshape(B, IDROWS, C)
    mesh = plsc.VectorSubcoreMesh(core_axis_name="c", subcore_axis_name="s")
    return pl.kernel(
        _seg_body,
        out_type=jax.ShapeDtypeStruct((B, S, D), jnp.float32),
        mesh=mesh,
        compiler_params=pltpu.CompilerParams(use_tc_tiling_on_sc=False),
        scratch_types=[
            pltpu.VMEM((CPT, C), jnp.int32),
            pltpu.VMEM((C, D), jnp.float32),
            pltpu.VMEM((C, D), jnp.float32),
            pltpu.VMEM((C, D), jnp.float32),
            pltpu.VMEM((ZROWS, D), jnp.float32),
            pltpu.VMEM_SHARED((S, D), jnp.float32),
            pltpu.SemaphoreType.DMA,
            pltpu.SemaphoreType.DMA,
            pltpu.SemaphoreType.DMA,
            pltpu.SemaphoreType.DMA,
            pltpu.SemaphoreType.DMA,
        ],
    )(data, ids32)


# speedup vs baseline: 1.0027x; 1.0027x over previous
import jax
import jax.numpy as jnp
from jax import lax
from jax.experimental import pallas as pl
from jax.experimental.pallas import tpu as pltpu
from jax.experimental.pallas import tpu_sc as plsc

B = 4
N = 160000
D = 128
S = 10000
NC = 2
NS = 16

C = 80
CPT = N // (NS * C)
ROWS_PER_TILE = C * CPT
IDROWS = N // C
SEG_PER_TILE = S // NS
ZROWS = 25


def _seg_body(data_hbm, ids_hbm, out_hbm, idx_v, chunk_a, chunk_b, chunk_c,
              zero_v, acc_sh, sem_la, sem_lb, sem_lc, sem_i, sem_z):
    c = lax.axis_index("c")
    s = lax.axis_index("s")

    def _zfill(k, carry):
        zero_v[k // (D // 16), pl.ds((k % (D // 16)) * 16, 16)] = jnp.zeros(
            (16,), jnp.float32)
        return carry
    lax.fori_loop(0, ZROWS * (D // 16), _zfill, 0)

    row0 = s * ROWS_PER_TILE
    idrow0 = s * CPT

    def _load(batch, j, buf, sem):
        pltpu.async_copy(
            data_hbm.at[batch, pl.ds(row0 + j * C, C)], buf, sem)

    def _wait_load(batch, buf, sem):
        pltpu.make_async_copy(
            data_hbm.at[batch, pl.ds(row0, C)], buf, sem).wait()

    def _prefire(batch):
        pltpu.async_copy(ids_hbm.at[batch, pl.ds(idrow0, CPT)], idx_v, sem_i)
        _load(batch, 0, chunk_a, sem_la)
        _load(batch, 1, chunk_b, sem_lb)

    def _zero_fire_drain(batch):
        zero_d = [
            pltpu.async_copy(
                zero_v, acc_sh.at[pl.ds(s * SEG_PER_TILE + k * ZROWS, ZROWS)],
                sem_z)
            for k in range(SEG_PER_TILE // ZROWS)
        ]
        for d in zero_d:
            d.wait()
        pltpu.make_async_copy(
            ids_hbm.at[batch, pl.ds(idrow0, CPT)], idx_v, sem_i).wait()

    def _main_loop(batch):
        def _scat(j, buf):
            pltpu.sync_copy(buf, acc_sh.at[idx_v.at[j]], add=True)

        _wait_load(batch, chunk_a, sem_la)
        _load(batch, 2, chunk_c, sem_lc)
        _scat(0, chunk_a)
        _wait_load(batch, chunk_b, sem_lb)
        _load(batch, 3, chunk_a, sem_la)
        _scat(1, chunk_b)

        def _tri(t, carry):
            j = 3 * t + 2
            _wait_load(batch, chunk_c, sem_lc)
            _load(batch, j + 2, chunk_b, sem_lb)
            _scat(j, chunk_c)

            _wait_load(batch, chunk_a, sem_la)
            _load(batch, j + 3, chunk_c, sem_lc)
            _scat(j + 1, chunk_a)

            _wait_load(batch, chunk_b, sem_lb)
            _load(batch, j + 4, chunk_a, sem_la)
            _scat(j + 2, chunk_b)
            return carry
        lax.fori_loop(0, (CPT - 5) // 3, _tri, 0)

        _wait_load(batch, chunk_c, sem_lc)
        _load(batch, CPT - 1, chunk_b, sem_lb)
        _scat(CPT - 3, chunk_c)

        _wait_load(batch, chunk_a, sem_la)
        _scat(CPT - 2, chunk_a)

        _wait_load(batch, chunk_b, sem_lb)
        _scat(CPT - 1, chunk_b)

    def _copy_out(batch):
        pltpu.sync_copy(
            acc_sh.at[pl.ds(s * SEG_PER_TILE, SEG_PER_TILE)],
            out_hbm.at[batch, pl.ds(s * SEG_PER_TILE, SEG_PER_TILE)])

    b0 = c * (B // NC)
    _prefire(b0)
    _zero_fire_drain(b0)
    plsc.subcore_barrier()
    _main_loop(b0)
    plsc.subcore_barrier()

    _prefire(b0 + 1)
    _copy_out(b0)
    _zero_fire_drain(b0 + 1)
    plsc.subcore_barrier()
    _main_loop(b0 + 1)
    plsc.subcore_barrier()
    _copy_out(b0 + 1)


@jax.jit
def kernel(data, segment_ids):
    ids32 = segment_ids.astype(jnp.int32).reshape(B, IDROWS, C)
    mesh = plsc.VectorSubcoreMesh(core_axis_name="c", subcore_axis_name="s")
    return pl.kernel(
        _seg_body,
        out_type=jax.ShapeDtypeStruct((B, S, D), jnp.float32),
        mesh=mesh,
        compiler_params=pltpu.CompilerParams(use_tc_tiling_on_sc=False),
        scratch_types=[
            pltpu.VMEM((CPT, C), jnp.int32),
            pltpu.VMEM((C, D), jnp.float32),
            pltpu.VMEM((C, D), jnp.float32),
            pltpu.VMEM((C, D), jnp.float32),
            pltpu.VMEM((ZROWS, D), jnp.float32),
            pltpu.VMEM_SHARED((S, D), jnp.float32),
            pltpu.SemaphoreType.DMA,
            pltpu.SemaphoreType.DMA,
            pltpu.SemaphoreType.DMA,
            pltpu.SemaphoreType.DMA,
            pltpu.SemaphoreType.DMA,
        ],
    )(data, ids32)
